# baseline (speedup 1.0000x reference)
import functools

import jax
import jax.numpy as jnp
from jax import lax
from jax.experimental import pallas as pl
from jax.experimental.pallas import tpu as pltpu
from jax.experimental.pallas import tpu_sc as plsc

T = 2048
D = 768
F = 3072
E = 8
K = 2
N = T * K
BM = 512
R = N + E * BM
NB = R // BM

NW = 32
CHUNK = T // NW
D2 = D // 2

def _sc_dispatch_body(x_hbm, pos_hbm, xs_hbm, idx0_v, idx1_v, rows_v, sem):
    wid = lax.axis_index("s") * 2 + lax.axis_index("c")
    base = wid * CHUNK
    pltpu.sync_copy(pos_hbm.at[0, pl.ds(base, CHUNK)], idx0_v)
    pltpu.sync_copy(pos_hbm.at[1, pl.ds(base, CHUNK)], idx1_v)
    pltpu.sync_copy(x_hbm.at[pl.ds(base, CHUNK)], rows_v)
    pltpu.async_copy(rows_v, xs_hbm.at[idx0_v], sem).wait()
    pltpu.async_copy(rows_v, xs_hbm.at[idx1_v], sem).wait()


@functools.lru_cache(maxsize=None)
def _sc_dispatch_kernel():
    mesh = plsc.VectorSubcoreMesh(core_axis_name="c", subcore_axis_name="s")
    return pl.kernel(
        _sc_dispatch_body,
        out_type=jax.ShapeDtypeStruct((R, D2), jnp.int32),
        mesh=mesh,
        scratch_types=[
            pltpu.VMEM((CHUNK,), jnp.int32),
            pltpu.VMEM((CHUNK,), jnp.int32),
            pltpu.VMEM((CHUNK, D2), jnp.int32),
            pltpu.SemaphoreType.DMA,
        ],
    )


def _sc_dispatch(x, pos_kt):
    xw = jax.lax.bitcast_convert_type(
        x.reshape(T, D2, 2), jnp.int32)
    xsw = _sc_dispatch_kernel()(xw, pos_kt)
    return jax.lax.bitcast_convert_type(xsw, jnp.bfloat16).reshape(R, D)


def _sc_combine_body(ys_hbm, pos_hbm, y0_hbm, y1_hbm,
                     idx0_v, idx1_v, rows0_v, rows1_v, sem):
    wid = lax.axis_index("s") * 2 + lax.axis_index("c")
    base = wid * CHUNK
    pltpu.sync_copy(pos_hbm.at[0, pl.ds(base, CHUNK)], idx0_v)
    pltpu.sync_copy(pos_hbm.at[1, pl.ds(base, CHUNK)], idx1_v)
    cp0 = pltpu.async_copy(ys_hbm.at[idx0_v], rows0_v, sem)
    cp1 = pltpu.async_copy(ys_hbm.at[idx1_v], rows1_v, sem)
    cp0.wait()
    cp1.wait()
    pltpu.sync_copy(rows0_v, y0_hbm.at[pl.ds(base, CHUNK)])
    pltpu.sync_copy(rows1_v, y1_hbm.at[pl.ds(base, CHUNK)])


@functools.lru_cache(maxsize=None)
def _sc_combine_kernel():
    mesh = plsc.VectorSubcoreMesh(core_axis_name="c", subcore_axis_name="s")
    return pl.kernel(
        _sc_combine_body,
        out_type=(jax.ShapeDtypeStruct((T, D2), jnp.int32),
                  jax.ShapeDtypeStruct((T, D2), jnp.int32)),
        mesh=mesh,
        scratch_types=[
            pltpu.VMEM((CHUNK,), jnp.int32),
            pltpu.VMEM((CHUNK,), jnp.int32),
            pltpu.VMEM((CHUNK, D2), jnp.int32),
            pltpu.VMEM((CHUNK, D2), jnp.int32),
            pltpu.SemaphoreType.DMA,
        ],
    )


def _sc_combine(ys, pos_kt):
    ysw = jax.lax.bitcast_convert_type(
        ys.reshape(R, D2, 2), jnp.int32)
    y0w, y1w = _sc_combine_kernel()(ysw, pos_kt)
    y0 = jax.lax.bitcast_convert_type(y0w, jnp.bfloat16).reshape(T, D)
    y1 = jax.lax.bitcast_convert_type(y1w, jnp.bfloat16).reshape(T, D)
    return y0, y1


def _router_body(x_ref, wg_ref, pos_ref, gat_ref, meta_ref):
    logits = jnp.dot(x_ref[...], wg_ref[...], preferred_element_type=jnp.float32)
    p = jnp.exp(logits - jnp.max(logits, axis=1, keepdims=True))

    lane = jax.lax.broadcasted_iota(jnp.int32, (T, E), 1)
    v1 = jnp.max(p, axis=1, keepdims=True)
    i1 = jnp.min(jnp.where(p == v1, lane, E), axis=1, keepdims=True)
    m1 = lane == i1
    p2 = jnp.where(m1, -1.0, p)
    v2 = jnp.max(p2, axis=1, keepdims=True)
    i2 = jnp.min(jnp.where(p2 == v2, lane, E), axis=1, keepdims=True)
    m2 = lane == i2
    denom = v1 + v2
    g0 = v1 / denom
    g1 = v2 / denom

    cnt = m1.astype(jnp.float32) + m2.astype(jnp.float32)
    pre = cnt
    k = 1
    while k < T:
        pre = pre + jnp.concatenate(
            [jnp.zeros((k, E), jnp.float32), pre[:T - k]], axis=0)
        k *= 2
    excl = jnp.concatenate(
        [jnp.zeros((1, E), jnp.float32), pre[:T - 1]], axis=0)

    counts = pre[T - 1:T, :]
    padded = jnp.floor((counts + (BM - 1)) / BM) * BM
    po = padded
    k = 1
    while k < E:
        po = po + jnp.concatenate(
            [jnp.zeros((1, k), jnp.float32), po[:, :E - k]], axis=1)
        k *= 2
    pad_off = po - padded

    rank0 = jnp.sum(jnp.where(m1, excl, 0.0), axis=1, keepdims=True)
    rank1 = jnp.sum(jnp.where(m2, excl, 0.0), axis=1, keepdims=True)
    off0 = jnp.sum(jnp.where(m1, pad_off, 0.0), axis=1, keepdims=True)
    off1 = jnp.sum(jnp.where(m2, pad_off, 0.0), axis=1, keepdims=True)
    pos0 = (off0 + rank0).astype(jnp.int32)
    pos1 = (off1 + rank1).astype(jnp.int32)

    posmat = jnp.where(lane == 0, pos0, 0) + jnp.where(lane == 1, pos1, 0)
    gatmat = jnp.where(lane == 0, g0, 0.0) + jnp.where(lane == 1, g1, 0.0)
    pos_ref[...] = jnp.transpose(posmat)[:K, :]
    gat_ref[...] = jnp.transpose(gatmat)[:K, :]

    bstart = (jax.lax.broadcasted_iota(jnp.int32, (NB, E), 0) * BM
              ).astype(jnp.float32)
    cmp = (jnp.broadcast_to(pad_off, (NB, E)) <= bstart).astype(jnp.int32)
    exp_id = jnp.sum(cmp, axis=1, keepdims=True) - 1
    lane_b = jax.lax.broadcasted_iota(jnp.int32, (NB, E), 1)
    onehot_b = (lane_b == exp_id).astype(jnp.float32)
    pad_end = pad_off + counts
    sel_end = jnp.sum(onehot_b * jnp.broadcast_to(pad_end, (NB, E)),
                      axis=1, keepdims=True)
    active = (bstart[:, 0:1] < sel_end).astype(jnp.int32)
    lastb = jnp.sum(active) - 1
    biota = jax.lax.broadcasted_iota(jnp.int32, (NB, 1), 0)
    db = jnp.where(active > 0, biota, lastb)
    meta_ref[:, 0:1] = exp_id
    meta_ref[:, 1:2] = active
    meta_ref[:, 2:3] = db
    meta_ref[:, 3:4] = jnp.zeros((NB, 1), jnp.int32)


def _router(x, Wg):
    return pl.pallas_call(
        _router_body,
        out_shape=(jax.ShapeDtypeStruct((K, T), jnp.int32),
                   jax.ShapeDtypeStruct((K, T), jnp.float32),
                   jax.ShapeDtypeStruct((NB, 4), jnp.int32)),
    )(x, Wg)


def _ffn_body(meta_ref, xs_ref, w1_ref, b1_ref, w2_ref, b2_ref, ys_ref):
    b = pl.program_id(0)
    active = meta_ref[b, 1]

    @pl.when(active > 0)
    def _():
        xb = xs_ref[...].astype(jnp.float32)
        h = jnp.dot(xb, w1_ref[0], preferred_element_type=jnp.float32)
        h = h + b1_ref[0]
        h = h * jax.nn.sigmoid(h)
        y = jnp.dot(h, w2_ref[0], preferred_element_type=jnp.float32)
        ys_ref[...] = (y + b2_ref[0]).astype(jnp.bfloat16)


def _grouped_ffn(xs, w1, b1, w2, b2, meta):
    grid_spec = pltpu.PrefetchScalarGridSpec(
        num_scalar_prefetch=1,
        grid=(NB,),
        in_specs=[
            pl.BlockSpec((BM, D), lambda b, m: (m[b, 2], 0)),
            pl.BlockSpec((1, D, F), lambda b, m: (m[b, 0], 0, 0)),
            pl.BlockSpec((1, 1, F), lambda b, m: (m[b, 0], 0, 0)),
            pl.BlockSpec((1, F, D), lambda b, m: (m[b, 0], 0, 0)),
            pl.BlockSpec((1, 1, D), lambda b, m: (m[b, 0], 0, 0)),
        ],
        out_specs=pl.BlockSpec((BM, D), lambda b, m: (m[b, 2], 0)),
    )
    return pl.pallas_call(
        _ffn_body,
        grid_spec=grid_spec,
        out_shape=jax.ShapeDtypeStruct((R, D), jnp.bfloat16),
    )(meta, xs, w1, b1.reshape(E, 1, F), w2, b2.reshape(E, 1, D))


def kernel(x, Wg, w1, b1, w2, b2):
    pos_kt, gat_kt, meta = _router(x, Wg)

    xs = _sc_dispatch(x.astype(jnp.bfloat16), pos_kt)

    ys = _grouped_ffn(xs, w1, b1, w2, b2, meta)

    y0, y1 = _sc_combine(ys, pos_kt)
    return gat_kt[0][:, None] * y0 + gat_kt[1][:, None] * y1


# speedup vs baseline: 3.2019x; 3.2019x over previous
import functools

import jax
import jax.numpy as jnp
from jax import lax
from jax.experimental import pallas as pl
from jax.experimental.pallas import tpu as pltpu
from jax.experimental.pallas import tpu_sc as plsc

T = 2048
D = 768
F = 3072
E = 8
K = 2
N = T * K
BM = 512
R = N + E * BM
NB = R // BM

NW = 32
CHUNK = T // NW
D2 = D // 2

def _sc_dispatch_body(x_hbm, pos_hbm, xs_hbm, idx0_v, idx1_v, rows_v, sem):
    wid = lax.axis_index("s") * 2 + lax.axis_index("c")
    base = wid * CHUNK
    pltpu.sync_copy(pos_hbm.at[0, pl.ds(base, CHUNK)], idx0_v)
    pltpu.sync_copy(pos_hbm.at[1, pl.ds(base, CHUNK)], idx1_v)
    pltpu.sync_copy(x_hbm.at[pl.ds(base, CHUNK)], rows_v)
    pltpu.async_copy(rows_v, xs_hbm.at[idx0_v], sem).wait()
    pltpu.async_copy(rows_v, xs_hbm.at[idx1_v], sem).wait()


@functools.lru_cache(maxsize=None)
def _sc_dispatch_kernel():
    mesh = plsc.VectorSubcoreMesh(core_axis_name="c", subcore_axis_name="s")
    return pl.kernel(
        _sc_dispatch_body,
        out_type=jax.ShapeDtypeStruct((R, D), jnp.float32),
        mesh=mesh,
        scratch_types=[
            pltpu.VMEM((CHUNK,), jnp.int32),
            pltpu.VMEM((CHUNK,), jnp.int32),
            pltpu.VMEM((CHUNK, D), jnp.float32),
            pltpu.SemaphoreType.DMA,
        ],
    )


def _sc_dispatch(x, pos_kt):
    return _sc_dispatch_kernel()(x, pos_kt)


def _sc_combine_body(ys_hbm, pos_hbm, y0_hbm, y1_hbm,
                     idx0_v, idx1_v, rows0_v, rows1_v, sem):
    wid = lax.axis_index("s") * 2 + lax.axis_index("c")
    base = wid * CHUNK
    pltpu.sync_copy(pos_hbm.at[0, pl.ds(base, CHUNK)], idx0_v)
    pltpu.sync_copy(pos_hbm.at[1, pl.ds(base, CHUNK)], idx1_v)
    cp0 = pltpu.async_copy(ys_hbm.at[idx0_v], rows0_v, sem)
    cp1 = pltpu.async_copy(ys_hbm.at[idx1_v], rows1_v, sem)
    cp0.wait()
    cp1.wait()
    pltpu.sync_copy(rows0_v, y0_hbm.at[pl.ds(base, CHUNK)])
    pltpu.sync_copy(rows1_v, y1_hbm.at[pl.ds(base, CHUNK)])


@functools.lru_cache(maxsize=None)
def _sc_combine_kernel():
    mesh = plsc.VectorSubcoreMesh(core_axis_name="c", subcore_axis_name="s")
    return pl.kernel(
        _sc_combine_body,
        out_type=(jax.ShapeDtypeStruct((T, D), jnp.float32),
                  jax.ShapeDtypeStruct((T, D), jnp.float32)),
        mesh=mesh,
        scratch_types=[
            pltpu.VMEM((CHUNK,), jnp.int32),
            pltpu.VMEM((CHUNK,), jnp.int32),
            pltpu.VMEM((CHUNK, D), jnp.float32),
            pltpu.VMEM((CHUNK, D), jnp.float32),
            pltpu.SemaphoreType.DMA,
        ],
    )


def _sc_combine(ys, pos_kt):
    return _sc_combine_kernel()(ys, pos_kt)


def _router_body(x_ref, wg_ref, pos_ref, gat_ref, meta_ref):
    logits = jnp.dot(x_ref[...], wg_ref[...], preferred_element_type=jnp.float32)
    p = jnp.exp(logits - jnp.max(logits, axis=1, keepdims=True))

    lane = jax.lax.broadcasted_iota(jnp.int32, (T, E), 1)
    v1 = jnp.max(p, axis=1, keepdims=True)
    i1 = jnp.min(jnp.where(p == v1, lane, E), axis=1, keepdims=True)
    m1 = lane == i1
    p2 = jnp.where(m1, -1.0, p)
    v2 = jnp.max(p2, axis=1, keepdims=True)
    i2 = jnp.min(jnp.where(p2 == v2, lane, E), axis=1, keepdims=True)
    m2 = lane == i2
    denom = v1 + v2
    g0 = v1 / denom
    g1 = v2 / denom

    cnt = m1.astype(jnp.float32) + m2.astype(jnp.float32)
    pre = cnt
    k = 1
    while k < T:
        pre = pre + jnp.concatenate(
            [jnp.zeros((k, E), jnp.float32), pre[:T - k]], axis=0)
        k *= 2
    excl = jnp.concatenate(
        [jnp.zeros((1, E), jnp.float32), pre[:T - 1]], axis=0)

    counts = pre[T - 1:T, :]
    padded = jnp.floor((counts + (BM - 1)) / BM) * BM
    po = padded
    k = 1
    while k < E:
        po = po + jnp.concatenate(
            [jnp.zeros((1, k), jnp.float32), po[:, :E - k]], axis=1)
        k *= 2
    pad_off = po - padded

    rank0 = jnp.sum(jnp.where(m1, excl, 0.0), axis=1, keepdims=True)
    rank1 = jnp.sum(jnp.where(m2, excl, 0.0), axis=1, keepdims=True)
    off0 = jnp.sum(jnp.where(m1, pad_off, 0.0), axis=1, keepdims=True)
    off1 = jnp.sum(jnp.where(m2, pad_off, 0.0), axis=1, keepdims=True)
    pos0 = (off0 + rank0).astype(jnp.int32)
    pos1 = (off1 + rank1).astype(jnp.int32)

    posmat = jnp.where(lane == 0, pos0, 0) + jnp.where(lane == 1, pos1, 0)
    gatmat = jnp.where(lane == 0, g0, 0.0) + jnp.where(lane == 1, g1, 0.0)
    pos_ref[...] = jnp.transpose(posmat)[:K, :]
    gat_ref[...] = jnp.transpose(gatmat)[:K, :]

    bstart = (jax.lax.broadcasted_iota(jnp.int32, (NB, E), 0) * BM
              ).astype(jnp.float32)
    cmp = (jnp.broadcast_to(pad_off, (NB, E)) <= bstart).astype(jnp.int32)
    exp_id = jnp.sum(cmp, axis=1, keepdims=True) - 1
    lane_b = jax.lax.broadcasted_iota(jnp.int32, (NB, E), 1)
    onehot_b = (lane_b == exp_id).astype(jnp.float32)
    pad_end = pad_off + counts
    sel_end = jnp.sum(onehot_b * jnp.broadcast_to(pad_end, (NB, E)),
                      axis=1, keepdims=True)
    active = (bstart[:, 0:1] < sel_end).astype(jnp.int32)
    lastb = jnp.sum(active) - 1
    biota = jax.lax.broadcasted_iota(jnp.int32, (NB, 1), 0)
    db = jnp.where(active > 0, biota, lastb)
    meta_ref[:, 0:1] = exp_id
    meta_ref[:, 1:2] = active
    meta_ref[:, 2:3] = db
    meta_ref[:, 3:4] = jnp.zeros((NB, 1), jnp.int32)


def _router(x, Wg):
    return pl.pallas_call(
        _router_body,
        out_shape=(jax.ShapeDtypeStruct((K, T), jnp.int32),
                   jax.ShapeDtypeStruct((K, T), jnp.float32),
                   jax.ShapeDtypeStruct((NB, 4), jnp.int32)),
    )(x, Wg)


def _ffn_body(meta_ref, xs_ref, w1_ref, b1_ref, w2_ref, b2_ref, ys_ref):
    b = pl.program_id(0)
    active = meta_ref[b, 1]

    @pl.when(active > 0)
    def _():
        xb = xs_ref[...]
        h = jnp.dot(xb, w1_ref[0], preferred_element_type=jnp.float32)
        h = h + b1_ref[0]
        h = h * jax.nn.sigmoid(h)
        y = jnp.dot(h, w2_ref[0], preferred_element_type=jnp.float32)
        ys_ref[...] = y + b2_ref[0]


def _grouped_ffn(xs, w1, b1, w2, b2, meta):
    grid_spec = pltpu.PrefetchScalarGridSpec(
        num_scalar_prefetch=1,
        grid=(NB,),
        in_specs=[
            pl.BlockSpec((BM, D), lambda b, m: (m[b, 2], 0)),
            pl.BlockSpec((1, D, F), lambda b, m: (m[b, 0], 0, 0)),
            pl.BlockSpec((1, 1, F), lambda b, m: (m[b, 0], 0, 0)),
            pl.BlockSpec((1, F, D), lambda b, m: (m[b, 0], 0, 0)),
            pl.BlockSpec((1, 1, D), lambda b, m: (m[b, 0], 0, 0)),
        ],
        out_specs=pl.BlockSpec((BM, D), lambda b, m: (m[b, 2], 0)),
    )
    return pl.pallas_call(
        _ffn_body,
        grid_spec=grid_spec,
        out_shape=jax.ShapeDtypeStruct((R, D), jnp.float32),
    )(meta, xs, w1, b1.reshape(E, 1, F), w2, b2.reshape(E, 1, D))


def kernel(x, Wg, w1, b1, w2, b2):
    pos_kt, gat_kt, meta = _router(x, Wg)

    xs = _sc_dispatch(x, pos_kt)

    ys = _grouped_ffn(xs, w1, b1, w2, b2, meta)

    y0, y1 = _sc_combine(ys, pos_kt)
    return gat_kt[0][:, None] * y0 + gat_kt[1][:, None] * y1


# speedup vs baseline: 3.6727x; 1.1470x over previous
import functools

import jax
import jax.numpy as jnp
from jax import lax
from jax.experimental import pallas as pl
from jax.experimental.pallas import tpu as pltpu
from jax.experimental.pallas import tpu_sc as plsc

T = 2048
D = 768
F = 3072
E = 8
K = 2
N = T * K
BM = 640
NB = -(-(N + E * (BM - 1)) // BM)
R = NB * BM

NW = 32
CHUNK = T // NW
D2 = D // 2

def _sc_dispatch_body(x_hbm, pos_hbm, xs_hbm, idx0_v, idx1_v, rows_v, sem):
    wid = lax.axis_index("s") * 2 + lax.axis_index("c")
    base = wid * CHUNK
    pltpu.sync_copy(pos_hbm.at[0, pl.ds(base, CHUNK)], idx0_v)
    pltpu.sync_copy(pos_hbm.at[1, pl.ds(base, CHUNK)], idx1_v)
    pltpu.sync_copy(x_hbm.at[pl.ds(base, CHUNK)], rows_v)
    pltpu.async_copy(rows_v, xs_hbm.at[idx0_v], sem).wait()
    pltpu.async_copy(rows_v, xs_hbm.at[idx1_v], sem).wait()


@functools.lru_cache(maxsize=None)
def _sc_dispatch_kernel():
    mesh = plsc.VectorSubcoreMesh(core_axis_name="c", subcore_axis_name="s")
    return pl.kernel(
        _sc_dispatch_body,
        out_type=jax.ShapeDtypeStruct((R, D), jnp.float32),
        mesh=mesh,
        scratch_types=[
            pltpu.VMEM((CHUNK,), jnp.int32),
            pltpu.VMEM((CHUNK,), jnp.int32),
            pltpu.VMEM((CHUNK, D), jnp.float32),
            pltpu.SemaphoreType.DMA,
        ],
    )


def _sc_dispatch(x, pos_kt):
    return _sc_dispatch_kernel()(x, pos_kt)


def _sc_combine_body(ys_hbm, pos_hbm, y0_hbm, y1_hbm,
                     idx0_v, idx1_v, rows0_v, rows1_v, sem):
    wid = lax.axis_index("s") * 2 + lax.axis_index("c")
    base = wid * CHUNK
    pltpu.sync_copy(pos_hbm.at[0, pl.ds(base, CHUNK)], idx0_v)
    pltpu.sync_copy(pos_hbm.at[1, pl.ds(base, CHUNK)], idx1_v)
    cp0 = pltpu.async_copy(ys_hbm.at[idx0_v], rows0_v, sem)
    cp1 = pltpu.async_copy(ys_hbm.at[idx1_v], rows1_v, sem)
    cp0.wait()
    cp1.wait()
    pltpu.sync_copy(rows0_v, y0_hbm.at[pl.ds(base, CHUNK)])
    pltpu.sync_copy(rows1_v, y1_hbm.at[pl.ds(base, CHUNK)])


@functools.lru_cache(maxsize=None)
def _sc_combine_kernel():
    mesh = plsc.VectorSubcoreMesh(core_axis_name="c", subcore_axis_name="s")
    return pl.kernel(
        _sc_combine_body,
        out_type=(jax.ShapeDtypeStruct((T, D), jnp.float32),
                  jax.ShapeDtypeStruct((T, D), jnp.float32)),
        mesh=mesh,
        scratch_types=[
            pltpu.VMEM((CHUNK,), jnp.int32),
            pltpu.VMEM((CHUNK,), jnp.int32),
            pltpu.VMEM((CHUNK, D), jnp.float32),
            pltpu.VMEM((CHUNK, D), jnp.float32),
            pltpu.SemaphoreType.DMA,
        ],
    )


def _sc_combine(ys, pos_kt):
    return _sc_combine_kernel()(ys, pos_kt)


def _router_body(x_ref, wg_ref, pos_ref, gat_ref, meta_ref):
    logits = jnp.dot(x_ref[...], wg_ref[...], preferred_element_type=jnp.float32)
    p = jnp.exp(logits - jnp.max(logits, axis=1, keepdims=True))

    lane = jax.lax.broadcasted_iota(jnp.int32, (T, E), 1)
    v1 = jnp.max(p, axis=1, keepdims=True)
    i1 = jnp.min(jnp.where(p == v1, lane, E), axis=1, keepdims=True)
    m1 = lane == i1
    p2 = jnp.where(m1, -1.0, p)
    v2 = jnp.max(p2, axis=1, keepdims=True)
    i2 = jnp.min(jnp.where(p2 == v2, lane, E), axis=1, keepdims=True)
    m2 = lane == i2
    denom = v1 + v2
    g0 = v1 / denom
    g1 = v2 / denom

    cnt = m1.astype(jnp.float32) + m2.astype(jnp.float32)
    pre = cnt
    k = 1
    while k < T:
        pre = pre + jnp.concatenate(
            [jnp.zeros((k, E), jnp.float32), pre[:T - k]], axis=0)
        k *= 2
    excl = jnp.concatenate(
        [jnp.zeros((1, E), jnp.float32), pre[:T - 1]], axis=0)

    counts = pre[T - 1:T, :]
    padded = jnp.floor((counts + (BM - 1)) / BM) * BM
    po = padded
    k = 1
    while k < E:
        po = po + jnp.concatenate(
            [jnp.zeros((1, k), jnp.float32), po[:, :E - k]], axis=1)
        k *= 2
    pad_off = po - padded

    rank0 = jnp.sum(jnp.where(m1, excl, 0.0), axis=1, keepdims=True)
    rank1 = jnp.sum(jnp.where(m2, excl, 0.0), axis=1, keepdims=True)
    off0 = jnp.sum(jnp.where(m1, pad_off, 0.0), axis=1, keepdims=True)
    off1 = jnp.sum(jnp.where(m2, pad_off, 0.0), axis=1, keepdims=True)
    pos0 = (off0 + rank0).astype(jnp.int32)
    pos1 = (off1 + rank1).astype(jnp.int32)

    posmat = jnp.where(lane == 0, pos0, 0) + jnp.where(lane == 1, pos1, 0)
    gatmat = jnp.where(lane == 0, g0, 0.0) + jnp.where(lane == 1, g1, 0.0)
    pos_ref[...] = jnp.transpose(posmat)[:K, :]
    gat_ref[...] = jnp.transpose(gatmat)[:K, :]

    bstart = (jax.lax.broadcasted_iota(jnp.int32, (NB, E), 0) * BM
              ).astype(jnp.float32)
    cmp = (jnp.broadcast_to(pad_off, (NB, E)) <= bstart).astype(jnp.int32)
    exp_id = jnp.sum(cmp, axis=1, keepdims=True) - 1
    lane_b = jax.lax.broadcasted_iota(jnp.int32, (NB, E), 1)
    onehot_b = (lane_b == exp_id).astype(jnp.float32)
    pad_end = pad_off + counts
    sel_end = jnp.sum(onehot_b * jnp.broadcast_to(pad_end, (NB, E)),
                      axis=1, keepdims=True)
    active = (bstart[:, 0:1] < sel_end).astype(jnp.int32)
    lastb = jnp.sum(active) - 1
    biota = jax.lax.broadcasted_iota(jnp.int32, (NB, 1), 0)
    db = jnp.where(active > 0, biota, lastb)
    meta_ref[:, 0:1] = exp_id
    meta_ref[:, 1:2] = active
    meta_ref[:, 2:3] = db
    meta_ref[:, 3:4] = jnp.zeros((NB, 1), jnp.int32)


def _router(x, Wg):
    return pl.pallas_call(
        _router_body,
        out_shape=(jax.ShapeDtypeStruct((K, T), jnp.int32),
                   jax.ShapeDtypeStruct((K, T), jnp.float32),
                   jax.ShapeDtypeStruct((NB, 4), jnp.int32)),
    )(x, Wg)


def _ffn_body(meta_ref, xs_ref, w1_ref, b1_ref, w2_ref, b2_ref, ys_ref):
    b = pl.program_id(0)
    active = meta_ref[b, 1]

    @pl.when(active > 0)
    def _():
        xb = xs_ref[...]
        h = jnp.dot(xb, w1_ref[0], preferred_element_type=jnp.float32)
        h = h + b1_ref[0]
        h = h * jax.nn.sigmoid(h)
        y = jnp.dot(h, w2_ref[0], preferred_element_type=jnp.float32)
        ys_ref[...] = y + b2_ref[0]


def _grouped_ffn(xs, w1, b1, w2, b2, meta):
    grid_spec = pltpu.PrefetchScalarGridSpec(
        num_scalar_prefetch=1,
        grid=(NB,),
        in_specs=[
            pl.BlockSpec((BM, D), lambda b, m: (m[b, 2], 0)),
            pl.BlockSpec((1, D, F), lambda b, m: (m[b, 0], 0, 0)),
            pl.BlockSpec((1, 1, F), lambda b, m: (m[b, 0], 0, 0)),
            pl.BlockSpec((1, F, D), lambda b, m: (m[b, 0], 0, 0)),
            pl.BlockSpec((1, 1, D), lambda b, m: (m[b, 0], 0, 0)),
        ],
        out_specs=pl.BlockSpec((BM, D), lambda b, m: (m[b, 2], 0)),
    )
    return pl.pallas_call(
        _ffn_body,
        grid_spec=grid_spec,
        out_shape=jax.ShapeDtypeStruct((R, D), jnp.float32),
    )(meta, xs, w1, b1.reshape(E, 1, F), w2, b2.reshape(E, 1, D))


def kernel(x, Wg, w1, b1, w2, b2):
    pos_kt, gat_kt, meta = _router(x, Wg)

    xs = _sc_dispatch(x, pos_kt)

    ys = _grouped_ffn(xs, w1, b1, w2, b2, meta)

    y0, y1 = _sc_combine(ys, pos_kt)
    return gat_kt[0][:, None] * y0 + gat_kt[1][:, None] * y1
